# baseline (speedup 1.0000x reference)
import functools

import jax
import jax.numpy as jnp
from jax import lax
from jax.experimental import pallas as pl
from jax.experimental.pallas import tpu as pltpu
from jax.experimental.pallas import tpu_sc as plsc

_S = 10000
_NW = 32
_SPW = 320
_S_PAD = _NW * _SPW
_CH = 256
_ROW_BLK = 2000


def _scorer_body(x_ref, w1t_ref, b1_ref, w2t_ref, b2_ref, e_ref):
    h = jnp.dot(x_ref[...], w1t_ref[...], preferred_element_type=jnp.float32)
    h = jnp.maximum(h + b1_ref[...], 0.0)
    s = jnp.dot(h, w2t_ref[...], preferred_element_type=jnp.float32)
    e = jnp.exp(s + b2_ref[0, 0])
    e_ref[...] = jnp.broadcast_to(e, e_ref.shape)


def _scores_exp(x, W1, b1, W2, b2):
    n, d = x.shape
    h = W1.shape[0]
    return pl.pallas_call(
        _scorer_body,
        grid=(n // _ROW_BLK,),
        in_specs=[
            pl.BlockSpec((_ROW_BLK, d), lambda i: (i, 0)),
            pl.BlockSpec((d, h), lambda i: (0, 0)),
            pl.BlockSpec((1, h), lambda i: (0, 0)),
            pl.BlockSpec((h, 1), lambda i: (0, 0)),
            pl.BlockSpec((1, 1), lambda i: (0, 0)),
        ],
        out_specs=pl.BlockSpec((_ROW_BLK, 16), lambda i: (i, 0)),
        out_shape=jax.ShapeDtypeStruct((n, 16), jnp.float32),
    )(x, W1.T, b1.reshape(1, h), W2.T, b2.reshape(1, 1))


def _sc_pool(x, e16, batch, bounds, n_rows):
    mesh = plsc.VectorSubcoreMesh(core_axis_name="c", subcore_axis_name="s")

    @functools.partial(
        pl.kernel,
        out_type=jax.ShapeDtypeStruct((_S_PAD, 128), jnp.float32),
        mesh=mesh,
        scratch_types=[
            pltpu.VMEM((2 * _CH, 128), jnp.float32),
            pltpu.VMEM((256, 128), jnp.float32),
            pltpu.VMEM((64, 128), jnp.float32),
            pltpu.VMEM((512 * 16,), jnp.float32),
            pltpu.VMEM((2 * _CH * 16,), jnp.float32),
            pltpu.VMEM((1024,), jnp.int32),
            pltpu.VMEM((256,), jnp.float32),
            pltpu.VMEM((64,), jnp.int32),
            pltpu.SemaphoreType.DMA,
            pltpu.SemaphoreType.DMA,
            pltpu.SemaphoreType.DMA,
        ],
    )
    def k(x_hbm, e_hbm, b_hbm, bounds_hbm, out_hbm,
          x_buf, out_a, out_b, d_buf, e_buf, b_buf, spill, bd_buf, sem_x,
          sem_e, sem_b):
        wid = lax.axis_index("s") * 2 + lax.axis_index("c")
        seg_lo = wid * _SPW
        pltpu.sync_copy(bounds_hbm, bd_buf.at[pl.ds(0, 48)])
        lo = bd_buf[pl.ds(wid, 16)][0]
        hi = bd_buf[pl.ds(wid + 1, 16)][0]
        a0 = (lo // 8) * 8
        nch = (hi - a0 + _CH - 1) // _CH
        zero16 = jnp.zeros((16,), jnp.float32)

        def dz_body(t, c):
            d_buf[pl.ds(t * 16, 16)] = zero16
            return c

        lax.fori_loop(0, _SPW, dz_body, 0)

        def a_dma_of(kk):
            a = a0 + kk * _CH
            return pl.multiple_of(jnp.minimum(a, n_rows - _CH), 8)

        def issue(kk, half):
            ad = a_dma_of(kk)
            dst = half * _CH
            pltpu.async_copy(x_hbm.at[pl.ds(ad, _CH)],
                             x_buf.at[pl.ds(dst, _CH)], sem_x)
            pltpu.async_copy(e_hbm.at[pl.ds(ad * 16, _CH * 16)],
                             e_buf.at[pl.ds(dst * 16, _CH * 16)], sem_e)
            pltpu.async_copy(b_hbm.at[pl.ds(ad, _CH)],
                             b_buf.at[pl.ds(dst, _CH)], sem_b)

        def wait_one():
            pltpu.make_async_copy(x_hbm.at[pl.ds(0, _CH)],
                                  x_buf.at[pl.ds(0, _CH)], sem_x).wait()
            pltpu.make_async_copy(e_hbm.at[pl.ds(0, _CH * 16)],
                                  e_buf.at[pl.ds(0, _CH * 16)], sem_e).wait()
            pltpu.make_async_copy(b_hbm.at[pl.ds(0, _CH)],
                                  b_buf.at[pl.ds(0, _CH)], sem_b).wait()

        issue(jnp.int32(0), jnp.int32(0))

        def flush(rprev, raccs, rd):
            @pl.when(rprev >= 0)
            def _():
                tp = rprev - seg_lo

                @pl.when(tp < 256)
                def _():
                    for j in range(8):
                        out_a[tp, pl.ds(16 * j, 16)] = raccs[j]

                @pl.when(tp >= 256)
                def _():
                    for j in range(8):
                        out_b[tp - 256, pl.ds(16 * j, 16)] = raccs[j]

                d_buf[pl.ds(tp * 16, 16)] = rd

        def row_step(off, rc):
            rprev, raccs, rd = rc
            bid = b_buf[pl.ds(off, 16)][0]
            evv = e_buf[pl.ds(off * 16, 16)]
            changed = bid != rprev

            @pl.when(changed)
            def _():
                flush(rprev, raccs, rd)

            keep = jnp.where(changed, 0.0, 1.0)
            kv = jnp.full((16,), keep, jnp.float32)
            new = tuple(raccs[j] * kv + evv * x_buf[off, pl.ds(16 * j, 16)]
                        for j in range(8))
            rd = rd * kv + evv
            return bid, new, rd

        def spill_store(accs, dd):
            for j in range(8):
                spill[pl.ds(16 * j, 16)] = accs[j]
            spill[pl.ds(128, 16)] = dd

        def spill_load():
            return (tuple(spill[pl.ds(16 * j, 16)] for j in range(8)),
                    spill[pl.ds(128, 16)])

        def chunk_body(kk, carry):
            prev, accs, dd = carry
            b = kk % 2
            issue(jnp.minimum(kk + 1, nch - 1), 1 - b)
            wait_one()
            a = a0 + kk * _CH
            base = a_dma_of(kk) - b * _CH
            lim = jnp.minimum(hi, a + _CH)
            start = jnp.maximum(lo, a)
            full = jnp.logical_and(start == a, lim == a + _CH)
            spill_store(accs, dd)

            def fast(prev0):
                boff = b * _CH
                sa, sd = spill_load()

                @pl.loop(0, _CH, init_carry=(prev0, sa, sd), unroll=8)
                def body(q, rc):
                    return row_step(boff + q, rc)

                fp, fa, fd = body
                spill_store(fa, fd)
                return fp

            def slow(prev0):
                sa, sd = spill_load()
                sp, sa, sd = lax.fori_loop(
                    start, lim, lambda r, rc: row_step(r - base, rc),
                    (prev0, sa, sd))
                spill_store(sa, sd)
                return sp

            prev = lax.cond(full, fast, slow, prev)
            accs, dd = spill_load()
            return prev, accs, dd

        init = (jnp.int32(-1), tuple(zero16 for _ in range(8)), zero16)
        prev_f, accs_f, dd_f = lax.fori_loop(0, nch, chunk_body, init)
        flush(prev_f, accs_f, dd_f)
        wait_one()

        def norm_a(t, c):
            dv = d_buf[pl.ds(t * 16, 16)]
            pos = dv > 0.0
            rv = jnp.where(pos, 1.0 / dv, zero16)
            for j in range(8):
                av = out_a[t, pl.ds(16 * j, 16)]
                out_a[t, pl.ds(16 * j, 16)] = jnp.where(pos, av * rv, zero16)
            return c

        def norm_b(t, c):
            dv = d_buf[pl.ds((t + 256) * 16, 16)]
            pos = dv > 0.0
            rv = jnp.where(pos, 1.0 / dv, zero16)
            for j in range(8):
                av = out_b[t, pl.ds(16 * j, 16)]
                out_b[t, pl.ds(16 * j, 16)] = jnp.where(pos, av * rv, zero16)
            return c

        lax.fori_loop(0, 256, norm_a, 0)
        lax.fori_loop(0, _SPW - 256, norm_b, 0)
        pltpu.sync_copy(out_a, out_hbm.at[pl.ds(seg_lo, 256)])
        pltpu.sync_copy(out_b, out_hbm.at[pl.ds(seg_lo + 256, _SPW - 256)])

    return k(x, e16, batch, bounds)


def kernel(x, batch, W1, b1, W2, b2):
    n, _ = x.shape
    e16 = _scores_exp(x, W1, b1, W2, b2)
    batch32 = batch.astype(jnp.int32)
    thresh = (jnp.arange(33, dtype=jnp.int32) * _SPW)[None, :]
    bounds = jnp.sum((batch32[:, None] < thresh).astype(jnp.int32),
                     axis=0, dtype=jnp.int32)
    bounds = jnp.concatenate(
        [bounds, jnp.full((15,), jnp.int32(n), dtype=jnp.int32)])
    out_pad = _sc_pool(x, e16.reshape(-1), batch32, bounds, n)
    return out_pad[:_S]


# speedup vs baseline: 1.4224x; 1.4224x over previous
import functools

import jax
import jax.numpy as jnp
from jax import lax
from jax.experimental import pallas as pl
from jax.experimental.pallas import tpu as pltpu
from jax.experimental.pallas import tpu_sc as plsc

_S = 10000
_NW = 32
_SPW = 320
_S_PAD = _NW * _SPW
_CH = 128
_ROW_BLK = 3200


def _scorer_body(x_ref, w1t_ref, b1_ref, w2_ref, b2_ref, e_ref):
    h = jnp.dot(x_ref[...], w1t_ref[...], preferred_element_type=jnp.float32)
    h = jnp.maximum(h + b1_ref[...], 0.0)
    s = jnp.dot(w2_ref[...], h.T, preferred_element_type=jnp.float32)
    e = jnp.exp(s + b2_ref[0, 0])
    e_ref[...] = e.reshape(e_ref.shape)


def _scores_exp(x, W1, b1, W2, b2):
    n, d = x.shape
    h = W1.shape[0]
    return pl.pallas_call(
        _scorer_body,
        grid=(n // _ROW_BLK,),
        in_specs=[
            pl.BlockSpec((_ROW_BLK, d), lambda i: (i, 0)),
            pl.BlockSpec((d, h), lambda i: (0, 0)),
            pl.BlockSpec((1, h), lambda i: (0, 0)),
            pl.BlockSpec((1, h), lambda i: (0, 0)),
            pl.BlockSpec((1, 1), lambda i: (0, 0)),
        ],
        out_specs=pl.BlockSpec((1, 1, _ROW_BLK), lambda i: (i, 0, 0)),
        out_shape=jax.ShapeDtypeStruct((n // _ROW_BLK, 1, _ROW_BLK),
                                       jnp.float32),
    )(x, W1.T, b1.reshape(1, h), W2, b2.reshape(1, 1))


def _sc_pool(x, e16, batch, bounds, n_rows):
    mesh = plsc.VectorSubcoreMesh(core_axis_name="c", subcore_axis_name="s")

    @functools.partial(
        pl.kernel,
        out_type=jax.ShapeDtypeStruct((_S_PAD, 128), jnp.float32),
        mesh=mesh,
        scratch_types=[
            pltpu.VMEM((2 * _CH, 128), jnp.float32),
            pltpu.VMEM((256, 128), jnp.float32),
            pltpu.VMEM((64, 128), jnp.float32),
            pltpu.VMEM((512 * 16,), jnp.float32),
            pltpu.VMEM((1024,), jnp.float32),
            pltpu.VMEM((1024,), jnp.int32),
            pltpu.VMEM((256,), jnp.float32),
            pltpu.VMEM((64,), jnp.int32),
            pltpu.SemaphoreType.DMA,
            pltpu.SemaphoreType.DMA,
            pltpu.SemaphoreType.DMA,
        ],
    )
    def k(x_hbm, e_hbm, b_hbm, bounds_hbm, out_hbm,
          x_buf, out_a, out_b, d_buf, e_buf, b_buf, spill, bd_buf, sem_x,
          sem_e, sem_b):
        wid = lax.axis_index("s") * 2 + lax.axis_index("c")
        seg_lo = wid * _SPW
        pltpu.sync_copy(bounds_hbm, bd_buf.at[pl.ds(0, 48)])
        lo = bd_buf[pl.ds(wid, 16)][0]
        hi = bd_buf[pl.ds(wid + 1, 16)][0]
        a0 = (lo // 8) * 8
        nch = (hi - a0 + _CH - 1) // _CH
        zero16 = jnp.zeros((16,), jnp.float32)

        def dz_body(t, c):
            d_buf[pl.ds(t * 16, 16)] = zero16
            return c

        lax.fori_loop(0, _SPW, dz_body, 0)

        def a_dma_of(kk):
            a = a0 + kk * _CH
            return pl.multiple_of(jnp.minimum(a, n_rows - _CH), 8)

        def issue(kk, half):
            ad = a_dma_of(kk)
            dst = half * _CH
            pltpu.async_copy(x_hbm.at[pl.ds(ad, _CH)],
                             x_buf.at[pl.ds(dst, _CH)], sem_x)
            pltpu.async_copy(e_hbm.at[pl.ds(ad, _CH)],
                             e_buf.at[pl.ds(dst, _CH)], sem_e)
            pltpu.async_copy(b_hbm.at[pl.ds(ad, _CH)],
                             b_buf.at[pl.ds(dst, _CH)], sem_b)

        def wait_one():
            pltpu.make_async_copy(x_hbm.at[pl.ds(0, _CH)],
                                  x_buf.at[pl.ds(0, _CH)], sem_x).wait()
            pltpu.make_async_copy(e_hbm.at[pl.ds(0, _CH)],
                                  e_buf.at[pl.ds(0, _CH)], sem_e).wait()
            pltpu.make_async_copy(b_hbm.at[pl.ds(0, _CH)],
                                  b_buf.at[pl.ds(0, _CH)], sem_b).wait()

        issue(jnp.int32(0), jnp.int32(0))

        def flush(rprev, raccs, rd):
            @pl.when(rprev >= 0)
            def _():
                tp = rprev - seg_lo

                @pl.when(tp < 256)
                def _():
                    for j in range(8):
                        out_a[tp, pl.ds(16 * j, 16)] = raccs[j]

                @pl.when(tp >= 256)
                def _():
                    for j in range(8):
                        out_b[tp - 256, pl.ds(16 * j, 16)] = raccs[j]

                d_buf[pl.ds(tp * 16, 16)] = rd

        def row_step(off, rc):
            rprev, raccs, rd = rc
            bid = b_buf[pl.ds(off, 16)][0]
            evv = jnp.full((16,), e_buf[pl.ds(off, 16)][0], jnp.float32)
            changed = bid != rprev

            @pl.when(changed)
            def _():
                flush(rprev, raccs, rd)

            keep = jnp.where(changed, 0.0, 1.0)
            kv = jnp.full((16,), keep, jnp.float32)
            new = tuple(raccs[j] * kv + evv * x_buf[off, pl.ds(16 * j, 16)]
                        for j in range(8))
            rd = rd * kv + evv
            return bid, new, rd

        def spill_store(accs, dd):
            for j in range(8):
                spill[pl.ds(16 * j, 16)] = accs[j]
            spill[pl.ds(128, 16)] = dd

        def spill_load():
            return (tuple(spill[pl.ds(16 * j, 16)] for j in range(8)),
                    spill[pl.ds(128, 16)])

        def chunk_body(kk, carry):
            prev, accs, dd = carry
            b = kk % 2
            issue(jnp.minimum(kk + 1, nch - 1), 1 - b)
            wait_one()
            a = a0 + kk * _CH
            base = a_dma_of(kk) - b * _CH
            lim = jnp.minimum(hi, a + _CH)
            start = jnp.maximum(lo, a)
            full = jnp.logical_and(start == a, lim == a + _CH)
            spill_store(accs, dd)

            def fast(prev0):
                boff = b * _CH
                sa, sd = spill_load()

                @pl.loop(0, _CH, init_carry=(prev0, sa, sd), unroll=8)
                def body(q, rc):
                    return row_step(boff + q, rc)

                fp, fa, fd = body
                spill_store(fa, fd)
                return fp

            def slow(prev0):
                sa, sd = spill_load()
                sp, sa, sd = lax.fori_loop(
                    start, lim, lambda r, rc: row_step(r - base, rc),
                    (prev0, sa, sd))
                spill_store(sa, sd)
                return sp

            prev = lax.cond(full, fast, slow, prev)
            accs, dd = spill_load()
            return prev, accs, dd

        init = (jnp.int32(-1), tuple(zero16 for _ in range(8)), zero16)
        prev_f, accs_f, dd_f = lax.fori_loop(0, nch, chunk_body, init)
        flush(prev_f, accs_f, dd_f)
        wait_one()

        def norm_a(t, c):
            dv = d_buf[pl.ds(t * 16, 16)]
            pos = dv > 0.0
            rv = jnp.where(pos, 1.0 / dv, zero16)
            for j in range(8):
                av = out_a[t, pl.ds(16 * j, 16)]
                out_a[t, pl.ds(16 * j, 16)] = jnp.where(pos, av * rv, zero16)
            return c

        def norm_b(t, c):
            dv = d_buf[pl.ds((t + 256) * 16, 16)]
            pos = dv > 0.0
            rv = jnp.where(pos, 1.0 / dv, zero16)
            for j in range(8):
                av = out_b[t, pl.ds(16 * j, 16)]
                out_b[t, pl.ds(16 * j, 16)] = jnp.where(pos, av * rv, zero16)
            return c

        lax.fori_loop(0, 256, norm_a, 0)
        lax.fori_loop(0, _SPW - 256, norm_b, 0)
        pltpu.sync_copy(out_a, out_hbm.at[pl.ds(seg_lo, 256)])
        pltpu.sync_copy(out_b, out_hbm.at[pl.ds(seg_lo + 256, _SPW - 256)])

    return k(x, e16, batch, bounds)


def kernel(x, batch, W1, b1, W2, b2):
    n, _ = x.shape
    e3 = _scores_exp(x, W1, b1, W2, b2)
    batch32 = batch.astype(jnp.int32)
    thresh = (jnp.arange(33, dtype=jnp.int32) * _SPW)[None, :]
    bounds = jnp.sum((batch32[:, None] < thresh).astype(jnp.int32),
                     axis=0, dtype=jnp.int32)
    bounds = jnp.concatenate(
        [bounds, jnp.full((15,), jnp.int32(n), dtype=jnp.int32)])
    out_pad = _sc_pool(x, e3.reshape(-1), batch32, bounds, n)
    return out_pad[:_S]


# speedup vs baseline: 1.8677x; 1.3131x over previous
import functools

import jax
import jax.numpy as jnp
from jax import lax
from jax.experimental import pallas as pl
from jax.experimental.pallas import tpu as pltpu
from jax.experimental.pallas import tpu_sc as plsc

_S = 10000
_NW = 32
_SPW = 320
_S_PAD = _NW * _SPW
_CH = 128
_ROW_BLK = 3200


def _scorer_body(x_ref, w1t_ref, b1_ref, w2_ref, b2_ref, e_ref):
    h = jnp.dot(x_ref[...], w1t_ref[...], preferred_element_type=jnp.float32)
    h = jnp.maximum(h + b1_ref[...], 0.0)
    s = jnp.dot(w2_ref[...], h.T, preferred_element_type=jnp.float32)
    e = jnp.exp(s + b2_ref[0, 0])
    e_ref[...] = e.reshape(e_ref.shape)


def _scores_exp(x, W1, b1, W2, b2):
    n, d = x.shape
    h = W1.shape[0]
    return pl.pallas_call(
        _scorer_body,
        grid=(n // _ROW_BLK,),
        in_specs=[
            pl.BlockSpec((_ROW_BLK, d), lambda i: (i, 0)),
            pl.BlockSpec((d, h), lambda i: (0, 0)),
            pl.BlockSpec((1, h), lambda i: (0, 0)),
            pl.BlockSpec((1, h), lambda i: (0, 0)),
            pl.BlockSpec((1, 1), lambda i: (0, 0)),
        ],
        out_specs=pl.BlockSpec((1, 1, _ROW_BLK), lambda i: (i, 0, 0)),
        out_shape=jax.ShapeDtypeStruct((n // _ROW_BLK, 1, _ROW_BLK),
                                       jnp.float32),
    )(x, W1.T, b1.reshape(1, h), W2, b2.reshape(1, 1))


def _sc_pool(x, e16, batch, bounds, n_rows):
    mesh = plsc.VectorSubcoreMesh(core_axis_name="c", subcore_axis_name="s")

    @functools.partial(
        pl.kernel,
        out_type=jax.ShapeDtypeStruct((_S_PAD, 128), jnp.float32),
        mesh=mesh,
        scratch_types=[
            pltpu.VMEM((2 * _CH, 128), jnp.float32),
            pltpu.VMEM((256, 128), jnp.float32),
            pltpu.VMEM((64, 128), jnp.float32),
            pltpu.VMEM((512 * 16,), jnp.float32),
            pltpu.VMEM((1024,), jnp.float32),
            pltpu.VMEM((1024,), jnp.int32),
            pltpu.VMEM((256,), jnp.float32),
            pltpu.VMEM((64,), jnp.int32),
            pltpu.SemaphoreType.DMA,
            pltpu.SemaphoreType.DMA,
            pltpu.SemaphoreType.DMA,
        ],
    )
    def k(x_hbm, e_hbm, b_hbm, bounds_hbm, out_hbm,
          x_buf, out_a, out_b, d_buf, e_buf, b_buf, spill, bd_buf, sem_x,
          sem_e, sem_b):
        wid = lax.axis_index("s") * 2 + lax.axis_index("c")
        seg_lo = wid * _SPW
        pltpu.sync_copy(bounds_hbm, bd_buf.at[pl.ds(0, 48)])
        lo = bd_buf[pl.ds(wid, 16)][0]
        hi = bd_buf[pl.ds(wid + 1, 16)][0]
        a0 = (lo // 8) * 8
        nch = (hi - a0 + _CH - 1) // _CH
        zero16 = jnp.zeros((16,), jnp.float32)

        def dz_body(t, c):
            d_buf[pl.ds(t * 16, 16)] = zero16
            return c

        lax.fori_loop(0, _SPW, dz_body, 0)

        def a_dma_of(kk):
            a = a0 + kk * _CH
            return pl.multiple_of(jnp.minimum(a, n_rows - _CH), 8)

        def issue(kk, half):
            ad = a_dma_of(kk)
            dst = half * _CH
            pltpu.async_copy(x_hbm.at[pl.ds(ad, _CH)],
                             x_buf.at[pl.ds(dst, _CH)], sem_x)
            pltpu.async_copy(e_hbm.at[pl.ds(ad, _CH)],
                             e_buf.at[pl.ds(dst, _CH)], sem_e)
            pltpu.async_copy(b_hbm.at[pl.ds(ad, _CH)],
                             b_buf.at[pl.ds(dst, _CH)], sem_b)

        def wait_one():
            pltpu.make_async_copy(x_hbm.at[pl.ds(0, _CH)],
                                  x_buf.at[pl.ds(0, _CH)], sem_x).wait()
            pltpu.make_async_copy(e_hbm.at[pl.ds(0, _CH)],
                                  e_buf.at[pl.ds(0, _CH)], sem_e).wait()
            pltpu.make_async_copy(b_hbm.at[pl.ds(0, _CH)],
                                  b_buf.at[pl.ds(0, _CH)], sem_b).wait()

        issue(jnp.int32(0), jnp.int32(0))

        def flush(rprev, raccs, rd):
            @pl.when(rprev >= 0)
            def _():
                tp = rprev - seg_lo

                @pl.when(tp < 256)
                def _():
                    for j in range(8):
                        out_a[tp, pl.ds(16 * j, 16)] = raccs[j]

                @pl.when(tp >= 256)
                def _():
                    for j in range(8):
                        out_b[tp - 256, pl.ds(16 * j, 16)] = raccs[j]

                d_buf[pl.ds(tp * 16, 16)] = rd

        def row_step(off, rc):
            rprev, raccs, rd = rc
            bid = b_buf[pl.ds(off, 16)][0]
            evv = jnp.full((16,), e_buf[pl.ds(off, 16)][0], jnp.float32)
            changed = bid != rprev

            @pl.when(changed)
            def _():
                flush(rprev, raccs, rd)

            keep = jnp.where(changed, 0.0, 1.0)
            kv = jnp.full((16,), keep, jnp.float32)
            new = tuple(raccs[j] * kv + evv * x_buf[off, pl.ds(16 * j, 16)]
                        for j in range(8))
            rd = rd * kv + evv
            return bid, new, rd

        def spill_store(accs, dd):
            for j in range(8):
                spill[pl.ds(16 * j, 16)] = accs[j]
            spill[pl.ds(128, 16)] = dd

        def spill_load():
            return (tuple(spill[pl.ds(16 * j, 16)] for j in range(8)),
                    spill[pl.ds(128, 16)])

        def chunk_body(kk, carry):
            prev, accs, dd = carry
            b = kk % 2
            issue(jnp.minimum(kk + 1, nch - 1), 1 - b)
            wait_one()
            a = a0 + kk * _CH
            base = a_dma_of(kk) - b * _CH
            lim = jnp.minimum(hi, a + _CH)
            start = jnp.maximum(lo, a)
            full = jnp.logical_and(start == a, lim == a + _CH)
            spill_store(accs, dd)

            def fast(prev0):
                boff = b * _CH
                sa, sd = spill_load()

                @pl.loop(0, _CH // 16, init_carry=(prev0, sa, sd))
                def body(gq, rc):
                    gprev, gaccs, gd = rc
                    goff = boff + gq * 16
                    bids = b_buf[pl.ds(goff, 16)]
                    evs = e_buf[pl.ds(goff, 16)]
                    for a in range(16):
                        bid = bids[a]
                        evv = jnp.full((16,), evs[a], jnp.float32)
                        changed = bid != gprev

                        @pl.when(changed)
                        def _(gprev=gprev, gaccs=gaccs, gd=gd):
                            flush(gprev, gaccs, gd)

                        keep = jnp.where(changed, 0.0, 1.0)
                        kv = jnp.full((16,), keep, jnp.float32)
                        gaccs = tuple(
                            gaccs[j] * kv
                            + evv * x_buf[goff + a, pl.ds(16 * j, 16)]
                            for j in range(8))
                        gd = gd * kv + evv
                        gprev = bid
                    return gprev, gaccs, gd

                fp, fa, fd = body
                spill_store(fa, fd)
                return fp

            def slow(prev0):
                sa, sd = spill_load()
                sp, sa, sd = lax.fori_loop(
                    start, lim, lambda r, rc: row_step(r - base, rc),
                    (prev0, sa, sd))
                spill_store(sa, sd)
                return sp

            prev = lax.cond(full, fast, slow, prev)
            accs, dd = spill_load()
            return prev, accs, dd

        init = (jnp.int32(-1), tuple(zero16 for _ in range(8)), zero16)
        prev_f, accs_f, dd_f = lax.fori_loop(0, nch, chunk_body, init)
        flush(prev_f, accs_f, dd_f)
        wait_one()

        def norm_a(t, c):
            dv = d_buf[pl.ds(t * 16, 16)]
            pos = dv > 0.0
            rv = jnp.where(pos, 1.0 / dv, zero16)
            for j in range(8):
                av = out_a[t, pl.ds(16 * j, 16)]
                out_a[t, pl.ds(16 * j, 16)] = jnp.where(pos, av * rv, zero16)
            return c

        def norm_b(t, c):
            dv = d_buf[pl.ds((t + 256) * 16, 16)]
            pos = dv > 0.0
            rv = jnp.where(pos, 1.0 / dv, zero16)
            for j in range(8):
                av = out_b[t, pl.ds(16 * j, 16)]
                out_b[t, pl.ds(16 * j, 16)] = jnp.where(pos, av * rv, zero16)
            return c

        lax.fori_loop(0, 256, norm_a, 0)
        lax.fori_loop(0, _SPW - 256, norm_b, 0)
        pltpu.sync_copy(out_a, out_hbm.at[pl.ds(seg_lo, 256)])
        pltpu.sync_copy(out_b, out_hbm.at[pl.ds(seg_lo + 256, _SPW - 256)])

    return k(x, e16, batch, bounds)


def kernel(x, batch, W1, b1, W2, b2):
    n, _ = x.shape
    e3 = _scores_exp(x, W1, b1, W2, b2)
    batch32 = batch.astype(jnp.int32)
    thresh = (jnp.arange(33, dtype=jnp.int32) * _SPW)[None, :]
    bounds = jnp.sum((batch32[:, None] < thresh).astype(jnp.int32),
                     axis=0, dtype=jnp.int32)
    bounds = jnp.concatenate(
        [bounds, jnp.full((15,), jnp.int32(n), dtype=jnp.int32)])
    out_pad = _sc_pool(x, e3.reshape(-1), batch32, bounds, n)
    return out_pad[:_S]
